# Initial kernel scaffold; baseline (speedup 1.0000x reference)
import jax
import jax.numpy as jnp
from jax.experimental import pallas as pl


def kernel(input, weight):
    raise NotImplementedError("write your pallas kernel here")


# speedup vs baseline: 2.9778x; 2.9778x over previous
import functools

import jax
import jax.numpy as jnp
from jax import lax
from jax.experimental import pallas as pl
from jax.experimental.pallas import tpu as pltpu
from jax.experimental.pallas import tpu_sc as plsc

NUM_WORKERS = 32
CHUNK = 128
D = 128


def _sc_gather(weight, idx3):
    n_chunks = idx3.shape[1]
    rows_per_worker = n_chunks * CHUNK
    total_rows = NUM_WORKERS * rows_per_worker
    mesh = plsc.VectorSubcoreMesh(core_axis_name="c", subcore_axis_name="s")

    @functools.partial(
        pl.kernel,
        mesh=mesh,
        out_type=jax.ShapeDtypeStruct((total_rows, D), jnp.float32),
        scratch_types=[
            pltpu.VMEM((n_chunks, CHUNK), jnp.int32),
            pltpu.VMEM((CHUNK, D), jnp.float32),
            pltpu.SemaphoreType.DMA,
        ],
    )
    def k(table_hbm, idx_hbm, out_hbm, idx_v, rows_v, sem):
        wid = lax.axis_index("s") * 2 + lax.axis_index("c")
        base = wid * rows_per_worker
        pltpu.sync_copy(idx_hbm.at[wid], idx_v)

        def body(j, carry):
            pltpu.async_copy(table_hbm.at[idx_v.at[j]], rows_v, sem).wait()
            pltpu.sync_copy(rows_v, out_hbm.at[pl.ds(base + j * CHUNK, CHUNK)])
            return carry

        lax.fori_loop(0, n_chunks, body, 0)

    return k(weight, idx3)


def kernel(input, weight):
    b, s = input.shape
    idx3 = input.astype(jnp.int32).reshape(NUM_WORKERS, (b * s) // (NUM_WORKERS * CHUNK), CHUNK)
    out = _sc_gather(weight, idx3)
    return out.reshape(b, s, D)


# speedup vs baseline: 3.3179x; 1.1142x over previous
import functools

import jax
import jax.numpy as jnp
from jax import lax
from jax.experimental import pallas as pl
from jax.experimental.pallas import tpu as pltpu
from jax.experimental.pallas import tpu_sc as plsc

NUM_WORKERS = 32
CHUNK = 128
D = 128
NBUF = 5


def _sc_gather(weight, idx3):
    n_chunks = idx3.shape[1]
    rows_per_worker = n_chunks * CHUNK
    total_rows = NUM_WORKERS * rows_per_worker
    n_groups = n_chunks // NBUF
    assert n_chunks % NBUF == 0 and n_groups >= 2
    mesh = plsc.VectorSubcoreMesh(core_axis_name="c", subcore_axis_name="s")

    @functools.partial(
        pl.kernel,
        mesh=mesh,
        out_type=jax.ShapeDtypeStruct((total_rows, D), jnp.float32),
        scratch_types=(
            [pltpu.VMEM((n_chunks, CHUNK), jnp.int32)]
            + [pltpu.VMEM((CHUNK, D), jnp.float32) for _ in range(NBUF)]
            + [pltpu.SemaphoreType.DMA for _ in range(2 * NBUF)]
        ),
    )
    def k(table_hbm, idx_hbm, out_hbm, idx_v, *rest):
        bufs = rest[:NBUF]
        gsem = rest[NBUF:2 * NBUF]
        psem = rest[2 * NBUF:]
        wid = lax.axis_index("s") * 2 + lax.axis_index("c")
        base = wid * rows_per_worker
        pltpu.sync_copy(idx_hbm.at[wid], idx_v)

        def gather_start(c, b):
            pltpu.async_copy(table_hbm.at[idx_v.at[c]], bufs[b], gsem[b])

        def gather_wait(c, b):
            pltpu.make_async_copy(table_hbm.at[idx_v.at[c]], bufs[b], gsem[b]).wait()

        def put_start(c, b):
            pltpu.async_copy(bufs[b], out_hbm.at[pl.ds(base + c * CHUNK, CHUNK)], psem[b])

        def put_wait(c, b):
            pltpu.make_async_copy(bufs[b], out_hbm.at[pl.ds(base + c * CHUNK, CHUNK)], psem[b]).wait()

        for b in range(NBUF):
            gather_start(b, b)

        def body(g, carry):
            c0 = g * NBUF
            for b in range(NBUF):
                gather_wait(c0 + b, b)
                put_start(c0 + b, b)
            for b in range(NBUF):
                put_wait(c0 + b, b)
                gather_start(c0 + b + NBUF, b)
            return carry

        lax.fori_loop(0, n_groups - 1, body, 0)

        c0 = (n_groups - 1) * NBUF
        for b in range(NBUF):
            gather_wait(c0 + b, b)
            put_start(c0 + b, b)
        for b in range(NBUF):
            put_wait(c0 + b, b)

    return k(weight, idx3)


def kernel(input, weight):
    b, s = input.shape
    idx3 = input.astype(jnp.int32).reshape(NUM_WORKERS, (b * s) // (NUM_WORKERS * CHUNK), CHUNK)
    out = _sc_gather(weight, idx3)
    return out.reshape(b, s, D)


# speedup vs baseline: 10.1824x; 3.0689x over previous
import functools

import jax
import jax.numpy as jnp
from jax import lax
from jax.experimental import pallas as pl
from jax.experimental.pallas import tpu as pltpu
from jax.experimental.pallas import tpu_sc as plsc

NUM_WORKERS = 32
CHUNK = 128
D = 128
NBUF = 5


def _sc_gather(weight, idx3):
    n_chunks = idx3.shape[1]
    rows_per_worker = n_chunks * CHUNK
    total_rows = NUM_WORKERS * rows_per_worker
    n_groups = n_chunks // NBUF
    assert n_chunks % NBUF == 0 and n_groups >= 2
    mesh = plsc.VectorSubcoreMesh(core_axis_name="c", subcore_axis_name="s")

    @functools.partial(
        pl.kernel,
        mesh=mesh,
        out_type=jax.ShapeDtypeStruct((total_rows, D), jnp.float32),
        scratch_types=(
            [pltpu.VMEM((n_chunks, CHUNK), jnp.int32)]
            + [pltpu.VMEM((CHUNK, D), jnp.float32) for _ in range(NBUF)]
            + [pltpu.SemaphoreType.DMA for _ in range(2 * NBUF)]
        ),
    )
    def k(table_hbm, idx_hbm, out_hbm, idx_v, *rest):
        bufs = rest[:NBUF]
        gsem = rest[NBUF:2 * NBUF]
        psem = rest[2 * NBUF:]
        wid = lax.axis_index("s") * 2 + lax.axis_index("c")
        base = wid * rows_per_worker
        pltpu.sync_copy(idx_hbm.at[wid], idx_v)

        def gather_start(c, b):
            pltpu.async_copy(table_hbm.at[idx_v.at[c]], bufs[b], gsem[b])

        def gather_wait(c, b):
            pltpu.make_async_copy(table_hbm.at[idx_v.at[c]], bufs[b], gsem[b]).wait()

        def put_start(c, b):
            pltpu.async_copy(bufs[b], out_hbm.at[pl.ds(base + c * CHUNK, CHUNK)], psem[b])

        def put_wait(c, b):
            pltpu.make_async_copy(bufs[b], out_hbm.at[pl.ds(base + c * CHUNK, CHUNK)], psem[b]).wait()

        for b in range(NBUF):
            gather_start(b, b)

        def body(g, carry):
            c0 = g * NBUF
            for b in range(NBUF):
                gather_wait(c0 + b, b)
                put_start(c0 + b, b)
            for b in range(NBUF):
                put_wait(c0 + b, b)
                gather_start(c0 + b + NBUF, b)
            return carry

        lax.fori_loop(0, n_groups - 1, body, 0)

        c0 = (n_groups - 1) * NBUF
        for b in range(NBUF):
            gather_wait(c0 + b, b)
            put_start(c0 + b, b)
        for b in range(NBUF):
            put_wait(c0 + b, b)

    return k(weight, idx3)


def kernel(input, weight):
    b, s = input.shape
    idx3 = input.T.astype(jnp.int32).reshape(NUM_WORKERS, (b * s) // (NUM_WORKERS * CHUNK), CHUNK)
    out = _sc_gather(weight, idx3)
    return out.reshape(s, b, D).transpose(1, 0, 2)


# speedup vs baseline: 10.2793x; 1.0095x over previous
import functools

import jax
import jax.numpy as jnp
from jax import lax
from jax.experimental import pallas as pl
from jax.experimental.pallas import tpu as pltpu
from jax.experimental.pallas import tpu_sc as plsc

NUM_WORKERS = 32
CHUNK = 64
D = 128
NBUF = 10


def _sc_gather(weight, idx3):
    n_chunks = idx3.shape[1]
    rows_per_worker = n_chunks * CHUNK
    total_rows = NUM_WORKERS * rows_per_worker
    n_groups = n_chunks // NBUF
    assert n_chunks % NBUF == 0 and n_groups >= 2
    mesh = plsc.VectorSubcoreMesh(core_axis_name="c", subcore_axis_name="s")

    @functools.partial(
        pl.kernel,
        mesh=mesh,
        out_type=jax.ShapeDtypeStruct((total_rows, D), jnp.float32),
        scratch_types=(
            [pltpu.VMEM((n_chunks, CHUNK), jnp.int32)]
            + [pltpu.VMEM((CHUNK, D), jnp.float32) for _ in range(NBUF)]
            + [pltpu.SemaphoreType.DMA for _ in range(2 * NBUF)]
        ),
    )
    def k(table_hbm, idx_hbm, out_hbm, idx_v, *rest):
        bufs = rest[:NBUF]
        gsem = rest[NBUF:2 * NBUF]
        psem = rest[2 * NBUF:]
        wid = lax.axis_index("s") * 2 + lax.axis_index("c")
        base = wid * rows_per_worker
        pltpu.sync_copy(idx_hbm.at[wid], idx_v)

        def gather_start(c, b):
            pltpu.async_copy(table_hbm.at[idx_v.at[c]], bufs[b], gsem[b])

        def gather_wait(c, b):
            pltpu.make_async_copy(table_hbm.at[idx_v.at[c]], bufs[b], gsem[b]).wait()

        def put_start(c, b):
            pltpu.async_copy(bufs[b], out_hbm.at[pl.ds(base + c * CHUNK, CHUNK)], psem[b])

        def put_wait(c, b):
            pltpu.make_async_copy(bufs[b], out_hbm.at[pl.ds(base + c * CHUNK, CHUNK)], psem[b]).wait()

        for b in range(NBUF):
            gather_start(b, b)

        def body(g, carry):
            c0 = g * NBUF
            for b in range(NBUF):
                gather_wait(c0 + b, b)
                put_start(c0 + b, b)
            for b in range(NBUF):
                put_wait(c0 + b, b)
                gather_start(c0 + b + NBUF, b)
            return carry

        lax.fori_loop(0, n_groups - 1, body, 0)

        c0 = (n_groups - 1) * NBUF
        for b in range(NBUF):
            gather_wait(c0 + b, b)
            put_start(c0 + b, b)
        for b in range(NBUF):
            put_wait(c0 + b, b)

    return k(weight, idx3)


def kernel(input, weight):
    b, s = input.shape
    idx3 = input.T.astype(jnp.int32).reshape(NUM_WORKERS, (b * s) // (NUM_WORKERS * CHUNK), CHUNK)
    out = _sc_gather(weight, idx3)
    return out.reshape(s, b, D).transpose(1, 0, 2)
